# baseline (speedup 1.0000x reference)
import functools

import jax
import jax.numpy as jnp
from jax import lax
from jax.experimental import pallas as pl
from jax.experimental.pallas import tpu as pltpu
from jax.experimental.pallas import tpu_sc as plsc

_MAX_DIST = 10.0
_GRID_RESOLUTION = 1.0
_BOX = 21
_R3 = _BOX * _BOX * _BOX
_NC = 2
_NS = 16
_ROWS_PER_SUB = 584
_ACC_ROWS = _ROWS_PER_SUB * _NS
_TRASH = _ACC_ROWS - 1
_CHUNK = 128


def _voxelize_body(ct_ref, vox_ref):
    x = ct_ref[0:1, :]
    y = ct_ref[1:2, :]
    z = ct_ref[2:3, :]
    gx = jnp.round((x + _MAX_DIST) / _GRID_RESOLUTION)
    gy = jnp.round((y + _MAX_DIST) / _GRID_RESOLUTION)
    gz = jnp.round((z + _MAX_DIST) / _GRID_RESOLUTION)
    hi = float(_BOX - 1)
    inb = ((gx >= 0.0) & (gx <= hi) & (gy >= 0.0) & (gy <= hi)
           & (gz >= 0.0) & (gz <= hi))
    gxc = jnp.clip(gx, 0.0, hi)
    gyc = jnp.clip(gy, 0.0, hi)
    gzc = jnp.clip(gz, 0.0, hi)
    v = (gxc * float(_BOX * _BOX) + gyc * float(_BOX) + gzc).astype(jnp.int32)
    vox_ref[...] = jnp.where(inb, v, _TRASH)


def _voxelize(coords_t):
    n = coords_t.shape[1]
    return pl.pallas_call(
        _voxelize_body,
        out_shape=jax.ShapeDtypeStruct((1, n), jnp.int32),
    )(coords_t)


def _scatter_body(vox_hbm, feats_hbm, zeros_hbm, out_hbm,
                  idx_v, feat_v, acc_sh):
    c = lax.axis_index("c")
    s = lax.axis_index("s")
    tile = c * _NS + s
    n_chunks = idx_v.shape[0]

    pltpu.sync_copy(zeros_hbm, acc_sh.at[pl.ds(s * _ROWS_PER_SUB, _ROWS_PER_SUB)])
    plsc.subcore_barrier()

    pltpu.sync_copy(vox_hbm.at[pl.ds(tile * n_chunks, n_chunks)], idx_v)

    for j in range(n_chunks):
        base = (tile * n_chunks + j) * _CHUNK
        pltpu.sync_copy(feats_hbm.at[pl.ds(base, _CHUNK)], feat_v)
        pltpu.sync_copy(feat_v, acc_sh.at[idx_v.at[j]], add=True)

    plsc.subcore_barrier()
    pltpu.sync_copy(acc_sh.at[pl.ds(s * _ROWS_PER_SUB, _ROWS_PER_SUB)],
                    out_hbm.at[c, pl.ds(s * _ROWS_PER_SUB, _ROWS_PER_SUB)])


def _scatter(vox2d, feats, zeros):
    f = feats.shape[1]
    n_chunks = vox2d.shape[0] // (_NC * _NS)
    mesh = plsc.VectorSubcoreMesh(core_axis_name="c", subcore_axis_name="s")
    k = pl.kernel(
        _scatter_body,
        out_type=jax.ShapeDtypeStruct((_NC, _ACC_ROWS, f), jnp.float32),
        mesh=mesh,
        compiler_params=pltpu.CompilerParams(use_tc_tiling_on_sc=False),
        scratch_types=[
            pltpu.VMEM((n_chunks, _CHUNK), jnp.int32),
            pltpu.VMEM((_CHUNK, f), jnp.float32),
            pltpu.VMEM_SHARED((_ACC_ROWS, f), jnp.float32),
        ],
    )
    return k(vox2d, feats, zeros)


def _assemble_body(p_ref, out_ref):
    i = pl.program_id(0)

    @pl.when(i == 0)
    def _():
        out_ref[...] = p_ref[0:1, :_R3, :] + p_ref[1:2, :_R3, :]

    @pl.when(i != 0)
    def _():
        out_ref[...] = jnp.zeros_like(out_ref)


def _assemble(partials, batches):
    f = partials.shape[2]
    return pl.pallas_call(
        _assemble_body,
        grid=(batches,),
        in_specs=[pl.BlockSpec((_NC, _ACC_ROWS, f), lambda i: (0, 0, 0))],
        out_specs=pl.BlockSpec((1, _R3, f), lambda i: (i, 0, 0)),
        out_shape=jax.ShapeDtypeStruct((batches, _R3, f), jnp.float32),
    )(partials)


def kernel(coords, features):
    b, n, _ = coords.shape
    f = features.shape[2]
    p = b * n
    assert p % (_NC * _NS * _CHUNK) == 0

    coords_t = coords.reshape(p, 3).T
    vox = _voxelize(coords_t)
    vox2d = vox.reshape(p // _CHUNK, _CHUNK)
    feats = features.reshape(p, f)
    zeros = jnp.zeros((_ROWS_PER_SUB, f), jnp.float32)
    partials = _scatter(vox2d, feats, zeros)
    grid = _assemble(partials, b)
    return grid.reshape(b, _BOX, _BOX, _BOX, f)


# speedup vs baseline: 1.0045x; 1.0045x over previous
import jax
import jax.numpy as jnp
from jax import lax
from jax.experimental import pallas as pl
from jax.experimental.pallas import tpu as pltpu
from jax.experimental.pallas import tpu_sc as plsc

_MAX_DIST = 10.0
_GRID_RESOLUTION = 1.0
_BOX = 21
_ZPAD = 24
_OUT_ROWS = _BOX * _BOX * _ZPAD
_NC = 2
_NS = 16
_ROWS_PER_SUB = 672
_ACC_ROWS = _ROWS_PER_SUB * _NS
_TRASH = _ACC_ROWS - 1
_CHUNK = 128


def _voxelize_body(ct_ref, vox_ref):
    x = ct_ref[0:1, :]
    y = ct_ref[1:2, :]
    z = ct_ref[2:3, :]
    gx = jnp.round((x + _MAX_DIST) / _GRID_RESOLUTION)
    gy = jnp.round((y + _MAX_DIST) / _GRID_RESOLUTION)
    gz = jnp.round((z + _MAX_DIST) / _GRID_RESOLUTION)
    hi = float(_BOX - 1)
    inb = ((gx >= 0.0) & (gx <= hi) & (gy >= 0.0) & (gy <= hi)
           & (gz >= 0.0) & (gz <= hi))
    gxc = jnp.clip(gx, 0.0, hi)
    gyc = jnp.clip(gy, 0.0, hi)
    gzc = jnp.clip(gz, 0.0, hi)
    v = (gxc * float(_BOX * _ZPAD) + gyc * float(_ZPAD) + gzc).astype(jnp.int32)
    vox_ref[...] = jnp.where(inb, v, _TRASH)


def _voxelize(coords_t):
    n = coords_t.shape[1]
    return pl.pallas_call(
        _voxelize_body,
        out_shape=jax.ShapeDtypeStruct((1, n), jnp.int32),
    )(coords_t)


def _scatter_body(vox_hbm, feats_hbm, zeros_hbm, out_hbm,
                  idx_v, feat_v, acc_sh):
    c = lax.axis_index("c")
    s = lax.axis_index("s")
    tile = c * _NS + s
    n_chunks = idx_v.shape[0]

    pltpu.sync_copy(zeros_hbm, acc_sh.at[pl.ds(s * _ROWS_PER_SUB, _ROWS_PER_SUB)])
    plsc.subcore_barrier()

    pltpu.sync_copy(vox_hbm.at[pl.ds(tile * n_chunks, n_chunks)], idx_v)

    for j in range(n_chunks):
        base = (tile * n_chunks + j) * _CHUNK
        pltpu.sync_copy(feats_hbm.at[pl.ds(base, _CHUNK)], feat_v)
        pltpu.sync_copy(feat_v, acc_sh.at[idx_v.at[j]], add=True)

    plsc.subcore_barrier()

    last = _OUT_ROWS - 15 * _ROWS_PER_SUB

    @pl.when(s < _NS - 1)
    def _():
        pltpu.sync_copy(
            acc_sh.at[pl.ds(s * _ROWS_PER_SUB, _ROWS_PER_SUB)],
            out_hbm.at[c, pl.ds(s * _ROWS_PER_SUB, _ROWS_PER_SUB)])

    @pl.when(s == _NS - 1)
    def _():
        pltpu.sync_copy(
            acc_sh.at[pl.ds((_NS - 1) * _ROWS_PER_SUB, last)],
            out_hbm.at[c, pl.ds((_NS - 1) * _ROWS_PER_SUB, last)])


def _scatter(vox2d, feats, zeros):
    f = feats.shape[1]
    n_chunks = vox2d.shape[0] // (_NC * _NS)
    mesh = plsc.VectorSubcoreMesh(core_axis_name="c", subcore_axis_name="s")
    k = pl.kernel(
        _scatter_body,
        out_type=jax.ShapeDtypeStruct((_NC, _OUT_ROWS, f), jnp.float32),
        mesh=mesh,
        compiler_params=pltpu.CompilerParams(use_tc_tiling_on_sc=False),
        scratch_types=[
            pltpu.VMEM((n_chunks, _CHUNK), jnp.int32),
            pltpu.VMEM((_CHUNK, f), jnp.float32),
            pltpu.VMEM_SHARED((_ACC_ROWS, f), jnp.float32),
        ],
    )
    return k(vox2d, feats, zeros)


def _zerofill_body(out_ref):
    out_ref[...] = jnp.zeros_like(out_ref)


def _zerofill(batches, f):
    return pl.pallas_call(
        _zerofill_body,
        grid=(batches,),
        out_specs=pl.BlockSpec((1, _BOX, _BOX, _BOX, f),
                               lambda i: (i, 0, 0, 0, 0)),
        out_shape=jax.ShapeDtypeStruct((batches, _BOX, _BOX, _BOX, f),
                                       jnp.float32),
    )()


def _batch0_body(grid_ref, p_ref, out_ref):
    del grid_ref
    v = p_ref[0, 0, :, :_BOX, :] + p_ref[1, 0, :, :_BOX, :]
    out_ref[...] = v[None, None]


def _batch0(grid, partials5):
    batches, _, _, _, f = grid.shape
    return pl.pallas_call(
        _batch0_body,
        grid=(_BOX,),
        in_specs=[
            pl.BlockSpec(memory_space=pltpu.MemorySpace.HBM),
            pl.BlockSpec((_NC, 1, _BOX, _ZPAD, f),
                         lambda i: (0, i, 0, 0, 0)),
        ],
        out_specs=pl.BlockSpec((1, 1, _BOX, _BOX, f),
                               lambda i: (0, i, 0, 0, 0)),
        out_shape=jax.ShapeDtypeStruct(grid.shape, jnp.float32),
        input_output_aliases={0: 0},
    )(grid, partials5)


def kernel(coords, features):
    b, n, _ = coords.shape
    f = features.shape[2]
    p = b * n
    assert p % (_NC * _NS * _CHUNK) == 0

    coords_t = coords.reshape(p, 3).T
    vox = _voxelize(coords_t)
    vox2d = vox.reshape(p // _CHUNK, _CHUNK)
    feats = features.reshape(p, f)
    zeros = jnp.zeros((_ROWS_PER_SUB, f), jnp.float32)
    partials = _scatter(vox2d, feats, zeros)
    partials5 = partials.reshape(_NC, _BOX, _BOX, _ZPAD, f)
    zero_grid = _zerofill(b, f)
    return _batch0(zero_grid, partials5)


# speedup vs baseline: 1.4597x; 1.4532x over previous
import jax
import jax.numpy as jnp
from jax import lax
from jax.experimental import pallas as pl
from jax.experimental.pallas import tpu as pltpu
from jax.experimental.pallas import tpu_sc as plsc

_MAX_DIST = 10.0
_GRID_RESOLUTION = 1.0
_BOX = 21
_ZPAD = 24
_OUT_ROWS = _BOX * _BOX * _ZPAD
_NC = 2
_NS = 16
_ROWS_PER_SUB = 672
_ACC_ROWS = _ROWS_PER_SUB * _NS
_TRASH = _ACC_ROWS - 1
_CHUNK = 128


def _voxelize_body(ct_ref, vox_ref):
    x = ct_ref[0:1, :]
    y = ct_ref[1:2, :]
    z = ct_ref[2:3, :]
    gx = jnp.round((x + _MAX_DIST) / _GRID_RESOLUTION)
    gy = jnp.round((y + _MAX_DIST) / _GRID_RESOLUTION)
    gz = jnp.round((z + _MAX_DIST) / _GRID_RESOLUTION)
    hi = float(_BOX - 1)
    inb = ((gx >= 0.0) & (gx <= hi) & (gy >= 0.0) & (gy <= hi)
           & (gz >= 0.0) & (gz <= hi))
    gxc = jnp.clip(gx, 0.0, hi)
    gyc = jnp.clip(gy, 0.0, hi)
    gzc = jnp.clip(gz, 0.0, hi)
    v = (gxc * float(_BOX * _ZPAD) + gyc * float(_ZPAD) + gzc).astype(jnp.int32)
    vox_ref[...] = jnp.where(inb, v, _TRASH)


def _voxelize(coords_t):
    n = coords_t.shape[1]
    return pl.pallas_call(
        _voxelize_body,
        out_shape=jax.ShapeDtypeStruct((1, n), jnp.int32),
    )(coords_t)


def _scatter_body(vox_hbm, feats_hbm, zeros_hbm, out_hbm,
                  idx_v, feat_v, acc_sh):
    c = lax.axis_index("c")
    s = lax.axis_index("s")
    tile = c * _NS + s
    n_chunks = idx_v.shape[0]

    pltpu.sync_copy(zeros_hbm, acc_sh.at[pl.ds(s * _ROWS_PER_SUB, _ROWS_PER_SUB)])
    plsc.subcore_barrier()

    pltpu.sync_copy(vox_hbm.at[pl.ds(tile * n_chunks, n_chunks)], idx_v)

    for j in range(n_chunks):
        base = (tile * n_chunks + j) * _CHUNK
        pltpu.sync_copy(feats_hbm.at[pl.ds(base, _CHUNK)], feat_v)
        pltpu.sync_copy(feat_v, acc_sh.at[idx_v.at[j]], add=True)

    plsc.subcore_barrier()

    last = _OUT_ROWS - 15 * _ROWS_PER_SUB

    @pl.when(s < _NS - 1)
    def _():
        pltpu.sync_copy(
            acc_sh.at[pl.ds(s * _ROWS_PER_SUB, _ROWS_PER_SUB)],
            out_hbm.at[c, pl.ds(s * _ROWS_PER_SUB, _ROWS_PER_SUB)])

    @pl.when(s == _NS - 1)
    def _():
        pltpu.sync_copy(
            acc_sh.at[pl.ds((_NS - 1) * _ROWS_PER_SUB, last)],
            out_hbm.at[c, pl.ds((_NS - 1) * _ROWS_PER_SUB, last)])


def _scatter(vox2d, feats, zeros):
    f = feats.shape[1]
    n_chunks = vox2d.shape[0] // (_NC * _NS)
    mesh = plsc.VectorSubcoreMesh(core_axis_name="c", subcore_axis_name="s")
    k = pl.kernel(
        _scatter_body,
        out_type=jax.ShapeDtypeStruct((_NC, _OUT_ROWS, f), jnp.float32),
        mesh=mesh,
        compiler_params=pltpu.CompilerParams(use_tc_tiling_on_sc=False),
        scratch_types=[
            pltpu.VMEM((n_chunks, _CHUNK), jnp.int32),
            pltpu.VMEM((_CHUNK, f), jnp.float32),
            pltpu.VMEM_SHARED((_ACC_ROWS, f), jnp.float32),
        ],
    )
    return k(vox2d, feats, zeros)


def _combine_body(p_ref, out_ref):
    v = p_ref[0] + p_ref[1]
    out_ref[...] = v[None, :, :, :_BOX, :]


def _combine(partials5):
    f = partials5.shape[4]
    return pl.pallas_call(
        _combine_body,
        out_shape=jax.ShapeDtypeStruct((1, _BOX, _BOX, _BOX, f), jnp.float32),
    )(partials5)


def kernel(coords, features):
    b, n, _ = coords.shape
    f = features.shape[2]
    p = b * n
    assert p % (_NC * _NS * _CHUNK) == 0

    coords_t = coords.reshape(p, 3).T
    vox = _voxelize(coords_t)
    vox2d = vox.reshape(p // _CHUNK, _CHUNK)
    feats = features.reshape(p, f)
    zeros = jnp.zeros((_ROWS_PER_SUB, f), jnp.float32)
    partials = _scatter(vox2d, feats, zeros)
    partials5 = partials.reshape(_NC, _BOX, _BOX, _ZPAD, f)
    batch0 = _combine(partials5)
    grid = jnp.zeros((b, _BOX, _BOX, _BOX, f), jnp.float32)
    return lax.dynamic_update_slice(grid, batch0, (0, 0, 0, 0, 0))
